# baseline (speedup 1.0000x reference)
import functools

import jax
import jax.numpy as jnp
from jax import lax
from jax.experimental import pallas as pl
from jax.experimental.pallas import tpu as pltpu
from jax.experimental.pallas import tpu_sc as plsc

DIM = 256
LANES = 16
NSLICE = DIM // LANES
CHUNK = 64
DG = 4
NBUF = 2
UNROLL = 4


@functools.partial(jax.jit, static_argnames=("n_rows",))
def _sc_tokenize(table, idx, vals, fc_w, fc_b, wv, cls, n_rows):
    info = plsc.get_sparse_core_info()
    nc, ns = info.num_cores, info.num_subcores
    nw = nc * ns
    rpw = n_rows // nw
    nch = rpw // CHUNK

    mesh = plsc.VectorSubcoreMesh(core_axis_name="c", subcore_axis_name="s")

    @functools.partial(
        pl.kernel,
        mesh=mesh,
        compiler_params=pltpu.CompilerParams(needs_layout_passes=False),
        out_type=jax.ShapeDtypeStruct(((n_rows + 1) * DIM,), jnp.float32),
        scratch_types=[
            pltpu.VMEM((rpw,), jnp.int32),
            pltpu.VMEM((rpw,), jnp.float32),
            pltpu.VMEM((4, DIM), jnp.float32),
            [pltpu.VMEM((CHUNK, DIM), jnp.float32)] * NBUF,
            [pltpu.VMEM((CHUNK * DIM,), jnp.float32)] * NBUF,
            pltpu.VMEM((DIM,), jnp.float32),
            [pltpu.SemaphoreType.DMA] * NBUF,
            [pltpu.SemaphoreType.DMA] * NBUF,
        ],
    )
    def k(table_hbm, idx_hbm, vals_hbm, fcw_hbm, fcb_hbm, wv_hbm, cls_hbm,
          out_flat, idx_v, vals_v, params_v, bufs, obufs, cls_v, gsems, wsems):
        wid = lax.axis_index("s") * nc + lax.axis_index("c")
        start = pl.multiple_of(wid * rpw, CHUNK)

        pltpu.sync_copy(fcw_hbm, params_v.at[0])
        pltpu.sync_copy(fcb_hbm, params_v.at[1])
        pltpu.sync_copy(wv_hbm, params_v.at[2])
        pltpu.sync_copy(cls_hbm, params_v.at[3])
        pltpu.sync_copy(idx_hbm.at[pl.ds(start, rpw)], idx_v)
        pltpu.sync_copy(vals_hbm.at[pl.ds(start, rpw)], vals_v)

        @pl.when(wid == 0)
        def _cls_row():
            for d in range(NSLICE):
                sl = pl.ds(d * LANES, LANES)
                cls_v[sl] = params_v[2, sl] + params_v[3, sl]
            pltpu.sync_copy(cls_v, out_flat.at[pl.ds(0, DIM)])

        gh = [None] * NBUF
        wh = [None] * NBUF
        gh[0] = pltpu.async_copy(
            table_hbm.at[idx_v.at[pl.ds(0, CHUNK)]], bufs[0], gsems[0])
        for c in range(nch):
            b = c % NBUF
            buf = bufs[b]
            obuf = obufs[b]
            gh[b].wait()
            if c + 1 < nch:
                nb = (c + 1) % NBUF
                if wh[nb] is not None:
                    wh[nb].wait()
                    wh[nb] = None
                gh[nb] = pltpu.async_copy(
                    table_hbm.at[idx_v.at[pl.ds((c + 1) * CHUNK, CHUNK)]],
                    bufs[nb], gsems[nb])
            base = c * CHUNK
            for dg in range(NSLICE // DG):
                sls = [pl.ds((dg * DG + g) * LANES, LANES) for g in range(DG)]
                ws = [params_v[0, sl] for sl in sls]
                bs = [params_v[1, sl] for sl in sls]
                wvs = [params_v[2, sl] for sl in sls]

                offs = [(dg * DG + g) * LANES for g in range(DG)]

                def body(i, buf=buf, obuf=obuf, sls=sls, ws=ws, bs=bs,
                         wvs=wvs, offs=offs, base=base):
                    s = plsc.load_gather(
                        vals_v, [jnp.full((LANES,), base + i, jnp.int32)])
                    for g in range(DG):
                        t = jnp.maximum(s * ws[g] + bs[g], 0.0) + wvs[g]
                        obuf[pl.ds(i * DIM + offs[g], LANES)] = (
                            buf[i, sls[g]] + t)

                plsc.parallel_loop(0, CHUNK, 1, unroll=UNROLL)(body)

            wh[b] = pltpu.async_copy(
                obuf,
                out_flat.at[pl.ds((1 + start + base) * DIM, CHUNK * DIM)],
                wsems[b])

        for b in range(NBUF):
            if wh[b] is not None:
                wh[b].wait()

    return k(table, idx, vals, fc_w, fc_b, wv, cls)


def kernel(indices, values, table, wv_omics, cls_token, fc_w, fc_b):
    L = indices.shape[0]
    out = _sc_tokenize(
        table, indices.astype(jnp.int32), values.astype(jnp.float32),
        fc_w.reshape(DIM), fc_b.reshape(DIM),
        wv_omics.reshape(DIM), cls_token.reshape(DIM), n_rows=L)
    return out.reshape(1, L + 1, DIM)


# speedup vs baseline: 2.7049x; 2.7049x over previous
import functools

import jax
import jax.numpy as jnp
from jax import lax
from jax.experimental import pallas as pl
from jax.experimental.pallas import tpu as pltpu
from jax.experimental.pallas import tpu_sc as plsc

DIM = 256
LANES = 16
NSLICE = DIM // LANES
CHUNK = 128
DG = 8
NBUF = 3
UNROLL = 2


@functools.partial(jax.jit, static_argnames=("n_rows",))
def _sc_tokenize(table, idx, vals, fc_w, fc_b, wv, cls, n_rows):
    info = plsc.get_sparse_core_info()
    nc, ns = info.num_cores, info.num_subcores
    nw = nc * ns
    rpw = n_rows // nw
    nch = rpw // CHUNK

    mesh = plsc.VectorSubcoreMesh(core_axis_name="c", subcore_axis_name="s")

    @functools.partial(
        pl.kernel,
        mesh=mesh,
        compiler_params=pltpu.CompilerParams(needs_layout_passes=False),
        out_type=jax.ShapeDtypeStruct((n_rows + 1, 1, DIM), jnp.float32),
        scratch_types=[
            pltpu.VMEM((rpw,), jnp.int32),
            pltpu.VMEM((rpw,), jnp.float32),
            pltpu.VMEM((4, DIM), jnp.float32),
            [pltpu.VMEM((CHUNK, DIM), jnp.float32)] * NBUF,
            pltpu.VMEM((DIM,), jnp.float32),
            [pltpu.SemaphoreType.DMA] * NBUF,
            [pltpu.SemaphoreType.DMA] * NBUF,
        ],
    )
    def k(table_hbm, idx_hbm, vals_hbm, fcw_hbm, fcb_hbm, wv_hbm, cls_hbm,
          out_3d, idx_v, vals_v, params_v, bufs, cls_v, gsems, wsems):
        out_hbm = out_3d.reshape(n_rows + 1, DIM)
        wid = lax.axis_index("s") * nc + lax.axis_index("c")
        start = pl.multiple_of(wid * rpw, CHUNK)

        pltpu.async_copy(idx_hbm.at[pl.ds(start, rpw)], idx_v,
                         wsems[0]).wait()
        hv = pltpu.async_copy(vals_hbm.at[pl.ds(start, rpw)], vals_v,
                              wsems[1])
        hps = [
            pltpu.async_copy(fcw_hbm, params_v.at[0], wsems[2]),
            pltpu.async_copy(fcb_hbm, params_v.at[1], wsems[2]),
            pltpu.async_copy(wv_hbm, params_v.at[2], wsems[2]),
            pltpu.async_copy(cls_hbm, params_v.at[3], wsems[2]),
        ]

        gh = [None] * NBUF
        wh = [None] * NBUF
        gh[0] = pltpu.async_copy(
            table_hbm.at[idx_v.at[pl.ds(0, CHUNK)]], bufs[0], gsems[0])

        hv.wait()
        for h in hps:
            h.wait()

        @pl.when(wid == 0)
        def _cls_row():
            for d in range(NSLICE):
                sl = pl.ds(d * LANES, LANES)
                cls_v[sl] = params_v[2, sl] + params_v[3, sl]
            pltpu.sync_copy(cls_v, out_hbm.at[0])

        for c in range(nch):
            b = c % NBUF
            buf = bufs[b]
            gh[b].wait()
            if c + 1 < nch:
                nb = (c + 1) % NBUF
                if wh[nb] is not None:
                    wh[nb].wait()
                    wh[nb] = None
                gh[nb] = pltpu.async_copy(
                    table_hbm.at[idx_v.at[pl.ds((c + 1) * CHUNK, CHUNK)]],
                    bufs[nb], gsems[nb])
            base = c * CHUNK
            for dg in range(NSLICE // DG):
                sls = [pl.ds((dg * DG + g) * LANES, LANES) for g in range(DG)]
                ws = [params_v[0, sl] for sl in sls]
                bs = [params_v[1, sl] for sl in sls]
                wvs = [params_v[2, sl] for sl in sls]

                def body(i, buf=buf, sls=sls, ws=ws, bs=bs, wvs=wvs,
                         base=base):
                    s = plsc.load_gather(
                        vals_v, [jnp.full((LANES,), base + i, jnp.int32)])
                    for g in range(DG):
                        t = jnp.maximum(s * ws[g] + bs[g], 0.0) + wvs[g]
                        buf[i, sls[g]] = buf[i, sls[g]] + t

                plsc.parallel_loop(0, CHUNK, 1, unroll=UNROLL)(body)

            wh[b] = pltpu.async_copy(
                buf, out_hbm.at[pl.ds(1 + start + base, CHUNK)], wsems[b])

        for b in range(NBUF):
            if wh[b] is not None:
                wh[b].wait()

    return k(table, idx, vals, fc_w, fc_b, wv, cls)


def kernel(indices, values, table, wv_omics, cls_token, fc_w, fc_b):
    L = indices.shape[0]
    out = _sc_tokenize(
        table, indices.astype(jnp.int32), values.astype(jnp.float32),
        fc_w.reshape(DIM), fc_b.reshape(DIM),
        wv_omics.reshape(DIM), cls_token.reshape(DIM), n_rows=L)
    return out.reshape(1, L + 1, DIM)
